# Initial kernel scaffold; baseline (speedup 1.0000x reference)
import jax
import jax.numpy as jnp
from jax.experimental import pallas as pl


def kernel(portfolio_value):
    raise NotImplementedError("write your pallas kernel here")


# speedup vs baseline: 11.0038x; 11.0038x over previous
import functools

import jax
import jax.numpy as jnp
from jax import lax
from jax.experimental import pallas as pl
from jax.experimental.pallas import tpu as pltpu
from jax.experimental.pallas import tpu_sc as plsc

N = 1048576
K = 52428
NC, NS, L = 2, 16, 16
NW = NC * NS
CHUNK = N // NW
NB1 = 4096
NB2 = 256

_mesh = plsc.VectorSubcoreMesh(
    core_axis_name="c", subcore_axis_name="s", num_cores=NC, num_subcores=NS)
_sc_params = pltpu.CompilerParams(needs_layout_passes=False)


def _keys(v):
    b = lax.bitcast_convert_type(v, jnp.int32)
    return b ^ (lax.shift_right_arithmetic(b, 31) & jnp.int32(0x7FFFFFFF))


@functools.partial(
    pl.kernel,
    out_type=jax.ShapeDtypeStruct((NW, NB1), jnp.int32),
    mesh=_mesh,
    compiler_params=_sc_params,
    scratch_types=[
        pltpu.VMEM((CHUNK,), jnp.float32),
        pltpu.VMEM((L * NB1,), jnp.int32),
        pltpu.VMEM((NB1,), jnp.int32),
    ],
)
def _hist1(x_hbm, out_hbm, vals, hist, folded):
    wid = lax.axis_index("s") * NC + lax.axis_index("c")
    pltpu.sync_copy(x_hbm.at[pl.ds(wid * CHUNK, CHUNK)], vals)
    zeros = jnp.zeros((L,), jnp.int32)

    def zero_body(i, _):
        hist[pl.ds(pl.multiple_of(i * L, L), L)] = zeros
        return 0
    lax.fori_loop(0, (L * NB1) // L, zero_body, 0, unroll=4)

    lanes = lax.iota(jnp.int32, L) * NB1
    ones = jnp.ones((L,), jnp.int32)

    def body(i, _):
        v = vals[pl.ds(pl.multiple_of(i * L, L), L)]
        bkt = lax.shift_right_arithmetic(_keys(v), 20) + jnp.int32(NB1 // 2)
        plsc.addupdate_scatter(hist, [lanes + bkt], ones)
        return 0
    lax.fori_loop(0, CHUNK // L, body, 0, unroll=4)

    def fold_body(g, _):
        off = pl.multiple_of(g * L, L)
        acc = hist[pl.ds(off, L)]
        for lane in range(1, L):
            acc = acc + hist[pl.ds(lane * NB1 + off, L)]
        folded[pl.ds(off, L)] = acc
        return 0
    lax.fori_loop(0, NB1 // L, fold_body, 0)
    pltpu.sync_copy(folded, out_hbm.at[wid])


def _merge1_body(h_ref, o_ref):
    m = jnp.sum(h_ref[...], axis=0)
    g = (lax.broadcasted_iota(jnp.int32, (32, 128), 0) * 128
         + lax.broadcasted_iota(jnp.int32, (32, 128), 1))

    def f(bound):
        return jnp.sum(jnp.where(g < bound, m, 0))

    def step(_, lohi):
        lo, hi = lohi
        mid = (lo + hi) // 2
        c = f(mid)
        return (jnp.where(c >= K, lo, mid), jnp.where(c >= K, mid, hi))

    _, hi = lax.fori_loop(0, 12, step, (jnp.int32(0), jnp.int32(NB1)))
    b1 = hi - 1
    c1 = f(b1)
    row = lax.broadcasted_iota(jnp.int32, (8, 128), 0)
    o_ref[...] = jnp.where(row == 0, b1, c1)


@functools.partial(
    pl.kernel,
    out_type=(
        jax.ShapeDtypeStruct((NW, NB2), jnp.int32),
        jax.ShapeDtypeStruct((NW, NB2), jnp.float32),
        jax.ShapeDtypeStruct((NW, L), jnp.float32),
    ),
    mesh=_mesh,
    compiler_params=_sc_params,
    scratch_types=[
        pltpu.VMEM((CHUNK,), jnp.float32),
        pltpu.VMEM((L * NB2,), jnp.int32),
        pltpu.VMEM((L * NB2,), jnp.float32),
        pltpu.VMEM((NB2,), jnp.int32),
        pltpu.VMEM((NB2,), jnp.float32),
        pltpu.VMEM((L,), jnp.float32),
        pltpu.VMEM((L,), jnp.int32),
    ],
)
def _hist2(x_hbm, scal_hbm, cnt_out, sum_out, s1_out,
           vals, hc, hs, fc, fs, accv, scal_v):
    wid = lax.axis_index("s") * NC + lax.axis_index("c")
    pltpu.sync_copy(x_hbm.at[pl.ds(wid * CHUNK, CHUNK)], vals)
    pltpu.sync_copy(scal_hbm, scal_v)
    b1 = scal_v[...][0]

    zi = jnp.zeros((L,), jnp.int32)
    zf = jnp.zeros((L,), jnp.float32)

    def zero_body(i, _):
        off = pl.multiple_of(i * L, L)
        hc[pl.ds(off, L)] = zi
        hs[pl.ds(off, L)] = zf
        return 0
    lax.fori_loop(0, (L * NB2) // L, zero_body, 0, unroll=4)

    lanes = lax.iota(jnp.int32, L) * NB2
    ones = jnp.ones((L,), jnp.int32)

    def body(i, acc):
        v = vals[pl.ds(pl.multiple_of(i * L, L), L)]
        key = _keys(v)
        bkt = lax.shift_right_arithmetic(key, 20) + jnp.int32(NB1 // 2)
        acc = acc + jnp.where(bkt < b1, v, 0.0)
        m = bkt == b1
        d = lax.shift_right_arithmetic(key, 12) & jnp.int32(0xFF)
        plsc.addupdate_scatter(hc, [lanes + d], ones, mask=m)
        plsc.addupdate_scatter(hs, [lanes + d], v, mask=m)
        return acc
    acc = lax.fori_loop(0, CHUNK // L, body, zf, unroll=2)
    accv[...] = acc

    def fold_body(g, _):
        off = pl.multiple_of(g * L, L)
        ac = hc[pl.ds(off, L)]
        asum = hs[pl.ds(off, L)]
        for lane in range(1, L):
            ac = ac + hc[pl.ds(lane * NB2 + off, L)]
            asum = asum + hs[pl.ds(lane * NB2 + off, L)]
        fc[pl.ds(off, L)] = ac
        fs[pl.ds(off, L)] = asum
        return 0
    lax.fori_loop(0, NB2 // L, fold_body, 0)
    pltpu.sync_copy(fc, cnt_out.at[wid])
    pltpu.sync_copy(fs, sum_out.at[wid])
    pltpu.sync_copy(accv, s1_out.at[wid])


def _final_body(hc_ref, hs_ref, s1_ref, scal_ref, o_ref):
    hc = jnp.sum(hc_ref[...], axis=0)
    hs = jnp.sum(hs_ref[...], axis=0)
    g = (lax.broadcasted_iota(jnp.int32, (2, 128), 0) * 128
         + lax.broadcasted_iota(jnp.int32, (2, 128), 1))
    c1 = scal_ref[1, 0]
    r1 = jnp.int32(K) - c1

    def f2(bound):
        return jnp.sum(jnp.where(g < bound, hc, 0))

    def step(_, lohi):
        lo, hi = lohi
        mid = (lo + hi) // 2
        c = f2(mid)
        return (jnp.where(c >= r1, lo, mid), jnp.where(c >= r1, mid, hi))

    _, hi = lax.fori_loop(0, 8, step, (jnp.int32(0), jnp.int32(NB2)))
    b2 = hi - 1
    c2 = f2(b2)
    r2 = (r1 - c2).astype(jnp.float32)
    s2below = jnp.sum(jnp.where(g < b2, hs, 0.0))
    cntb = jnp.sum(jnp.where(g == b2, hc, 0)).astype(jnp.float32)
    sumb = jnp.sum(jnp.where(g == b2, hs, 0.0))
    s1 = jnp.sum(s1_ref[...])
    total = s1 + s2below + r2 * (sumb / cntb)
    o_ref[...] = jnp.full((8, 128), total * (1.0 / K), jnp.float32)


def kernel(portfolio_value):
    h1 = _hist1(portfolio_value)
    scal = pl.pallas_call(
        _merge1_body,
        out_shape=jax.ShapeDtypeStruct((8, 128), jnp.int32),
    )(h1.reshape(NW, 32, 128))
    scal_c = scal[0, 0:16]
    hc, hs, s1p = _hist2(portfolio_value, scal_c)
    out = pl.pallas_call(
        _final_body,
        out_shape=jax.ShapeDtypeStruct((8, 128), jnp.float32),
    )(hc.reshape(NW, 2, 128), hs.reshape(NW, 2, 128),
      s1p.reshape(4, 128), scal)
    return out[0, 0]
